# baseline (speedup 1.0000x reference)
import jax
import jax.numpy as jnp
from jax import lax
from jax.experimental import pallas as pl
from jax.experimental.pallas import tpu as pltpu
from jax.experimental.pallas import tpu_sc as plsc

ROWS = 2097152
COLS = 4
BLOCK = 262144
NC = 2
NS = 16
NW = NC * NS
S = ROWS // NW


def _body(in_hbm, zeros_hbm, out_hbm):
    c = lax.axis_index("c")
    s = lax.axis_index("s")
    wid = s * NC + c
    in_off = wid * S
    k = wid // 4
    q = wid % 4
    out_off = k * (2 * BLOCK) + q * S
    zero_off = out_off + BLOCK
    pltpu.sync_copy(in_hbm.at[pl.ds(in_off, S)], out_hbm.at[pl.ds(out_off, S)])
    pltpu.sync_copy(zeros_hbm.at[:], out_hbm.at[pl.ds(zero_off, S)])


def kernel(psi):
    zeros = jnp.zeros((S, COLS), jnp.float32)
    mesh = plsc.VectorSubcoreMesh(core_axis_name="c", subcore_axis_name="s")
    run = pl.kernel(
        _body,
        out_type=jax.ShapeDtypeStruct((2 * ROWS, COLS), jnp.float32),
        mesh=mesh,
    )
    return run(psi, zeros)


# speedup vs baseline: 8.5539x; 8.5539x over previous
import jax
import jax.numpy as jnp
from jax import lax
from jax.experimental import pallas as pl
from jax.experimental.pallas import tpu as pltpu
from jax.experimental.pallas import tpu_sc as plsc

ROWS = 2097152
COLS = 4
IN_FLAT = ROWS * COLS
OUT_FLAT = 2 * IN_FLAT
CHUNK = 1048576
NC = 2
NS = 16
NW = NC * NS
S = IN_FLAT // NW


def _body(in_hbm, zeros_hbm, out_hbm):
    c = lax.axis_index("c")
    s = lax.axis_index("s")
    wid = s * NC + c
    in_off = wid * S
    k = wid // 4
    q = wid % 4
    out_off = k * (2 * CHUNK) + q * S
    zero_off = out_off + CHUNK
    pltpu.sync_copy(in_hbm.at[pl.ds(in_off, S)], out_hbm.at[pl.ds(out_off, S)])
    pltpu.sync_copy(zeros_hbm.at[:], out_hbm.at[pl.ds(zero_off, S)])


def kernel(psi):
    flat = psi.reshape(IN_FLAT)
    zeros = jnp.zeros((S,), jnp.float32)
    mesh = plsc.VectorSubcoreMesh(core_axis_name="c", subcore_axis_name="s")
    run = pl.kernel(
        _body,
        out_type=jax.ShapeDtypeStruct((OUT_FLAT,), jnp.float32),
        mesh=mesh,
        compiler_params=pltpu.CompilerParams(use_tc_tiling_on_sc=False),
    )
    return run(flat, zeros).reshape(2 * ROWS, COLS)
